# baseline (speedup 1.0000x reference)
import jax
import jax.numpy as jnp
from jax.experimental import pallas as pl
from jax.experimental.pallas import tpu as pltpu

_BS = 6400
_NBUF = 7


def _mlp_kernel(sc_hbm, hA_hbm, hB_hbm, w1_hbm, w1s_hbm, w2_hbm, w2b_hbm,
                wh_hbm, whb_hbm, outw_ref, outm_ref, sbuf, abuf, bbuf, w1buf, w1sbuf,
                w2buf, w2buf_b, whbuf, whbuf_b, in_sem, w_sem):
    i = pl.program_id(0)
    nb = pl.num_programs(0)

    _HB = _BS // 2

    def start(block, slot):
        pltpu.make_async_copy(
            hA_hbm.at[pl.ds(block * _BS, _HB), :],
            abuf.at[slot, pl.ds(0, _HB), :], in_sem.at[0, slot]).start()
        pltpu.make_async_copy(
            hA_hbm.at[pl.ds(block * _BS + _HB, _HB), :],
            abuf.at[slot, pl.ds(_HB, _HB), :], in_sem.at[1, slot]).start()
        pltpu.make_async_copy(
            hB_hbm.at[pl.ds(block * _BS, _HB), :],
            bbuf.at[slot, pl.ds(0, _HB), :], in_sem.at[2, slot]).start()
        pltpu.make_async_copy(
            hB_hbm.at[pl.ds(block * _BS + _HB, _HB), :],
            bbuf.at[slot, pl.ds(_HB, _HB), :], in_sem.at[3, slot]).start()
        pltpu.make_async_copy(
            sc_hbm.at[:, pl.ds(block * _BS, _BS)], sbuf.at[slot],
            in_sem.at[4, slot]).start()

    @pl.when(i == 0)
    def _prologue():
        for s in range(_NBUF):
            start(s, s)
        cps = [
            pltpu.make_async_copy(w1_hbm, w1buf, w_sem.at[0]),
            pltpu.make_async_copy(w1s_hbm, w1sbuf, w_sem.at[1]),
            pltpu.make_async_copy(w2_hbm, w2buf, w_sem.at[2]),
            pltpu.make_async_copy(w2b_hbm, w2buf_b, w_sem.at[3]),
            pltpu.make_async_copy(wh_hbm, whbuf, w_sem.at[4]),
            pltpu.make_async_copy(whb_hbm, whbuf_b, w_sem.at[5]),
        ]
        for c in cps:
            c.start()
        for c in cps:
            c.wait()

    slot = jax.lax.rem(i, _NBUF)
    H = 128

    pltpu.make_async_copy(
        sc_hbm.at[:, pl.ds(i * _BS, _BS)], sbuf.at[slot],
        in_sem.at[4, slot]).wait()

    u = jax.lax.dot_general(
        sbuf[slot], w1sbuf[0:5],
        (((0,), (0,)), ((), ())), preferred_element_type=jnp.float32)
    u = u + w1sbuf[5:6]

    pltpu.make_async_copy(
        hA_hbm.at[pl.ds(i * _BS, _HB), :],
        abuf.at[slot, pl.ds(0, _HB), :], in_sem.at[0, slot]).wait()
    pltpu.make_async_copy(
        hA_hbm.at[pl.ds(i * _BS + _HB, _HB), :],
        abuf.at[slot, pl.ds(_HB, _HB), :], in_sem.at[1, slot]).wait()
    pltpu.make_async_copy(
        hB_hbm.at[pl.ds(i * _BS, _HB), :],
        bbuf.at[slot, pl.ds(0, _HB), :], in_sem.at[2, slot]).wait()
    pltpu.make_async_copy(
        hB_hbm.at[pl.ds(i * _BS + _HB, _HB), :],
        bbuf.at[slot, pl.ds(_HB, _HB), :], in_sem.at[3, slot]).wait()

    f = jnp.dot(abuf[slot], w1buf[0:H],
                preferred_element_type=jnp.float32)
    f = f + jnp.dot(bbuf[slot], w1buf[H:2 * H],
                    preferred_element_type=jnp.float32)
    f = jnp.tanh(f + u)
    f = jnp.tanh(jnp.dot(f, w2buf[...],
                         preferred_element_type=jnp.float32)
                 + w2buf_b[0:1])
    r = (jnp.dot(f, whbuf[...], preferred_element_type=jnp.float32)
         + whbuf_b[0:1])
    outw_ref[...] = r[:, 0:1]
    outm_ref[...] = r[:, 1:2]

    @pl.when(i + _NBUF < nb)
    def _prefetch():
        start(i + _NBUF, slot)


def kernel(xi, h_A, h_B, E_val, I_val, L_val, q_val,
           W1, b1, W2, b2, Ww, bw, Wm, bm):
    B, H = h_A.shape
    D1 = W1.shape[1]
    D2 = W2.shape[1]

    scal = jnp.stack(
        [xi[:, 0], E_val[:, 0], I_val[:, 0], L_val[:, 0], q_val[:, 0]],
        axis=0)
    w1p = jnp.concatenate([
        W1[1:1 + H],
        W1[1 + H:1 + 2 * H],
    ], axis=0)
    w1sp = jnp.concatenate([
        W1[0:1], W1[1 + 2 * H:],
        b1.reshape(1, D1),
    ], axis=0)
    w2p = jnp.concatenate(
        [W2, jnp.zeros((H, H - D2), W2.dtype)], axis=1)
    w2bp = jnp.concatenate(
        [b2, jnp.zeros((H - D2,), b2.dtype)]).reshape(1, H)
    whp = jnp.concatenate([
        jnp.concatenate([Ww, Wm], axis=1),
        jnp.zeros((H - D2, 2), Ww.dtype),
    ], axis=0)
    whbp = jnp.concatenate([bw, bm]).reshape(1, 2)

    grid = (B // _BS,)
    row = lambda i: (i, 0)

    out = pl.pallas_call(
        _mlp_kernel,
        grid=grid,
        in_specs=[
            pl.BlockSpec(memory_space=pl.ANY),
            pl.BlockSpec(memory_space=pl.ANY),
            pl.BlockSpec(memory_space=pl.ANY),
            pl.BlockSpec(memory_space=pl.ANY),
            pl.BlockSpec(memory_space=pl.ANY),
            pl.BlockSpec(memory_space=pl.ANY),
            pl.BlockSpec(memory_space=pl.ANY),
            pl.BlockSpec(memory_space=pl.ANY),
            pl.BlockSpec(memory_space=pl.ANY),
        ],
        out_specs=[pl.BlockSpec((_BS, 1), row),
                   pl.BlockSpec((_BS, 1), row)],
        out_shape=[jax.ShapeDtypeStruct((B, 1), jnp.float32),
                   jax.ShapeDtypeStruct((B, 1), jnp.float32)],
        scratch_shapes=[
            pltpu.VMEM((_NBUF, 5, _BS), jnp.float32),
            pltpu.VMEM((_NBUF, _BS, H), jnp.float32),
            pltpu.VMEM((_NBUF, _BS, H), jnp.float32),
            pltpu.VMEM((2 * H, D1), jnp.float32),
            pltpu.VMEM((6, D1), jnp.float32),
            pltpu.VMEM((H, H), jnp.float32),
            pltpu.VMEM((1, H), jnp.float32),
            pltpu.VMEM((H, 2), jnp.float32),
            pltpu.VMEM((1, 2), jnp.float32),
            pltpu.SemaphoreType.DMA((5, _NBUF)),
            pltpu.SemaphoreType.DMA((6,)),
        ],
        compiler_params=pltpu.CompilerParams(
            dimension_semantics=("arbitrary",),
            vmem_limit_bytes=100 * 1024 * 1024),
    )(scal, h_A, h_B, w1p, w1sp, w2p, w2bp, whp, whbp)

    return (out[0], out[1])


# speedup vs baseline: 1.0028x; 1.0028x over previous
import jax
import jax.numpy as jnp
from jax.experimental import pallas as pl
from jax.experimental.pallas import tpu as pltpu

_BS = 6400
_NBUF = 6


def _mlp_kernel(sc_hbm, hA_hbm, hB_hbm, w1_hbm, w1s_hbm, w2_hbm, w2b_hbm,
                wh_hbm, whb_hbm, outw_ref, outm_ref, sbuf, abuf, bbuf, w1buf, w1sbuf,
                w2buf, w2buf_b, whbuf, whbuf_b, in_sem, w_sem):
    i = pl.program_id(0)
    nb = pl.num_programs(0)

    _HB = _BS // 2

    def start(block, slot):
        pltpu.make_async_copy(
            hA_hbm.at[pl.ds(block * _BS, _HB), :],
            abuf.at[slot, pl.ds(0, _HB), :], in_sem.at[0, slot]).start()
        pltpu.make_async_copy(
            hA_hbm.at[pl.ds(block * _BS + _HB, _HB), :],
            abuf.at[slot, pl.ds(_HB, _HB), :], in_sem.at[1, slot]).start()
        pltpu.make_async_copy(
            hB_hbm.at[pl.ds(block * _BS, _HB), :],
            bbuf.at[slot, pl.ds(0, _HB), :], in_sem.at[2, slot]).start()
        pltpu.make_async_copy(
            hB_hbm.at[pl.ds(block * _BS + _HB, _HB), :],
            bbuf.at[slot, pl.ds(_HB, _HB), :], in_sem.at[3, slot]).start()
        pltpu.make_async_copy(
            sc_hbm.at[:, pl.ds(block * _BS, _BS)], sbuf.at[slot],
            in_sem.at[4, slot]).start()

    @pl.when(i == 0)
    def _prologue():
        for s in range(_NBUF):
            start(s, s)
        cps = [
            pltpu.make_async_copy(w1_hbm, w1buf, w_sem.at[0]),
            pltpu.make_async_copy(w1s_hbm, w1sbuf, w_sem.at[1]),
            pltpu.make_async_copy(w2_hbm, w2buf, w_sem.at[2]),
            pltpu.make_async_copy(w2b_hbm, w2buf_b, w_sem.at[3]),
            pltpu.make_async_copy(wh_hbm, whbuf, w_sem.at[4]),
            pltpu.make_async_copy(whb_hbm, whbuf_b, w_sem.at[5]),
        ]
        for c in cps:
            c.start()
        for c in cps:
            c.wait()

    slot = jax.lax.rem(i, _NBUF)
    H = 128

    pltpu.make_async_copy(
        sc_hbm.at[:, pl.ds(i * _BS, _BS)], sbuf.at[slot],
        in_sem.at[4, slot]).wait()

    u = jax.lax.dot_general(
        sbuf[slot], w1sbuf[0:5],
        (((0,), (0,)), ((), ())), preferred_element_type=jnp.float32)
    u = u + w1sbuf[5:6]

    pltpu.make_async_copy(
        hA_hbm.at[pl.ds(i * _BS, _HB), :],
        abuf.at[slot, pl.ds(0, _HB), :], in_sem.at[0, slot]).wait()
    pltpu.make_async_copy(
        hA_hbm.at[pl.ds(i * _BS + _HB, _HB), :],
        abuf.at[slot, pl.ds(_HB, _HB), :], in_sem.at[1, slot]).wait()
    pltpu.make_async_copy(
        hB_hbm.at[pl.ds(i * _BS, _HB), :],
        bbuf.at[slot, pl.ds(0, _HB), :], in_sem.at[2, slot]).wait()
    pltpu.make_async_copy(
        hB_hbm.at[pl.ds(i * _BS + _HB, _HB), :],
        bbuf.at[slot, pl.ds(_HB, _HB), :], in_sem.at[3, slot]).wait()

    f = jnp.dot(abuf[slot], w1buf[0:H],
                preferred_element_type=jnp.float32)
    f = f + jnp.dot(bbuf[slot], w1buf[H:2 * H],
                    preferred_element_type=jnp.float32)
    f = jnp.tanh(f + u)
    f = jnp.tanh(jnp.dot(f, w2buf[...],
                         preferred_element_type=jnp.float32)
                 + w2buf_b[0:1])
    r = (jnp.dot(f, whbuf[...], preferred_element_type=jnp.float32)
         + whbuf_b[0:1])
    outw_ref[...] = r[:, 0:1]
    outm_ref[...] = r[:, 1:2]

    @pl.when(i + _NBUF < nb)
    def _prefetch():
        start(i + _NBUF, slot)


def kernel(xi, h_A, h_B, E_val, I_val, L_val, q_val,
           W1, b1, W2, b2, Ww, bw, Wm, bm):
    B, H = h_A.shape
    D1 = W1.shape[1]
    D2 = W2.shape[1]

    scal = jnp.stack(
        [xi[:, 0], E_val[:, 0], I_val[:, 0], L_val[:, 0], q_val[:, 0]],
        axis=0)
    w1p = jnp.concatenate([
        W1[1:1 + H],
        W1[1 + H:1 + 2 * H],
    ], axis=0)
    w1sp = jnp.concatenate([
        W1[0:1], W1[1 + 2 * H:],
        b1.reshape(1, D1),
    ], axis=0)
    w2p = jnp.concatenate(
        [W2, jnp.zeros((H, H - D2), W2.dtype)], axis=1)
    w2bp = jnp.concatenate(
        [b2, jnp.zeros((H - D2,), b2.dtype)]).reshape(1, H)
    whp = jnp.concatenate([
        jnp.concatenate([Ww, Wm], axis=1),
        jnp.zeros((H - D2, 2), Ww.dtype),
    ], axis=0)
    whbp = jnp.concatenate([bw, bm]).reshape(1, 2)

    grid = (B // _BS,)
    row = lambda i: (i, 0)

    out = pl.pallas_call(
        _mlp_kernel,
        grid=grid,
        in_specs=[
            pl.BlockSpec(memory_space=pl.ANY),
            pl.BlockSpec(memory_space=pl.ANY),
            pl.BlockSpec(memory_space=pl.ANY),
            pl.BlockSpec(memory_space=pl.ANY),
            pl.BlockSpec(memory_space=pl.ANY),
            pl.BlockSpec(memory_space=pl.ANY),
            pl.BlockSpec(memory_space=pl.ANY),
            pl.BlockSpec(memory_space=pl.ANY),
            pl.BlockSpec(memory_space=pl.ANY),
        ],
        out_specs=[pl.BlockSpec((_BS, 1), row),
                   pl.BlockSpec((_BS, 1), row)],
        out_shape=[jax.ShapeDtypeStruct((B, 1), jnp.float32),
                   jax.ShapeDtypeStruct((B, 1), jnp.float32)],
        scratch_shapes=[
            pltpu.VMEM((_NBUF, 5, _BS), jnp.float32),
            pltpu.VMEM((_NBUF, _BS, H), jnp.float32),
            pltpu.VMEM((_NBUF, _BS, H), jnp.float32),
            pltpu.VMEM((2 * H, D1), jnp.float32),
            pltpu.VMEM((6, D1), jnp.float32),
            pltpu.VMEM((H, H), jnp.float32),
            pltpu.VMEM((1, H), jnp.float32),
            pltpu.VMEM((H, 2), jnp.float32),
            pltpu.VMEM((1, 2), jnp.float32),
            pltpu.SemaphoreType.DMA((5, _NBUF)),
            pltpu.SemaphoreType.DMA((6,)),
        ],
        compiler_params=pltpu.CompilerParams(
            dimension_semantics=("arbitrary",),
            vmem_limit_bytes=100 * 1024 * 1024),
    )(scal, h_A, h_B, w1p, w1sp, w2p, w2bp, whp, whbp)

    return (out[0], out[1])
